# baseline (speedup 1.0000x reference)
import jax
import jax.numpy as jnp
from jax import lax
from jax.experimental import pallas as pl
from jax.experimental.pallas import tpu as pltpu
from jax.experimental.pallas import tpu_sc as plsc

NC = 2
NS = 16
NW = NC * NS

BATCH = 4096
HIST = 50
D = 128

B = BATCH * HIST
BPW = B // NW
CHUNK = 128
NCHUNK = BPW // CHUNK

_mesh = plsc.VectorSubcoreMesh(core_axis_name="c", subcore_axis_name="s")


NBUF = 5


def _gather_body(idx_hbm, table_hbm, out_hbm, idx_v,
                 rows0, rows1, rows2, rows3, rows4, gsem, wsem):
    wid = lax.axis_index("s") * NC + lax.axis_index("c")
    rows = (rows0, rows1, rows2, rows3, rows4)

    pltpu.sync_copy(idx_hbm.at[:, pl.ds(wid * CHUNK, CHUNK)], idx_v)

    def start_gather(ci, b):
        pltpu.make_async_copy(table_hbm.at[idx_v.at[ci]], rows[b],
                              gsem).start()

    def wait_gather(ci, b):
        pltpu.make_async_copy(table_hbm.at[idx_v.at[ci]], rows[b],
                              gsem).wait()

    def start_write(ci, b):
        off = ci * BATCH + wid * CHUNK
        pltpu.make_async_copy(rows[b], out_hbm.at[pl.ds(off, CHUNK)],
                              wsem).start()

    def wait_write(b):
        pltpu.make_async_copy(rows[b], out_hbm.at[pl.ds(0, CHUNK)],
                              wsem).wait()

    for b in range(NBUF):
        start_gather(b, b)

    def step(ci, b, refill):
        wait_gather(ci, b)
        start_write(ci, b)
        if refill is None:
            return
        wait_write(b)
        start_gather(refill, (b - 2) % NBUF)

    for b in range(2):
        wait_gather(b, b)
        start_write(b, b)
    for b in range(2, NBUF):
        step(b, b, b - 2 + NBUF)

    def group(g, _):
        for b in range(NBUF):
            ci = NBUF * g + b
            step(ci, b, ci - 2 + NBUF)
        return 0

    lax.fori_loop(1, NCHUNK // NBUF - 1, group, 0)

    for b in range(NBUF):
        ci = NCHUNK - NBUF + b
        step(ci, b, ci - 2 + NBUF if ci - 2 + NBUF < NCHUNK else None)
        if ci - 2 + NBUF >= NCHUNK:
            wait_write(b)
    wait_write(0)
    wait_write(1)


def _make_kernel(interpret=False):
    return pl.kernel(
        _gather_body,
        out_type=jax.ShapeDtypeStruct((B, D), jnp.float32),
        mesh=_mesh,
        scratch_types=[
            pltpu.VMEM((HIST, CHUNK), jnp.int32),
            pltpu.VMEM((CHUNK, D), jnp.float32),
            pltpu.VMEM((CHUNK, D), jnp.float32),
            pltpu.VMEM((CHUNK, D), jnp.float32),
            pltpu.VMEM((CHUNK, D), jnp.float32),
            pltpu.VMEM((CHUNK, D), jnp.float32),
            pltpu.SemaphoreType.DMA,
            pltpu.SemaphoreType.DMA,
        ],
        interpret=interpret,
    )


_gather_kernel = _make_kernel()


def kernel(x, GloVe):
    out = _gather_kernel(x.T.astype(jnp.int32), GloVe)
    return out.reshape(HIST, BATCH, D).transpose(1, 0, 2)
